# baseline (speedup 1.0000x reference)
import jax
import jax.numpy as jnp
from jax import lax
from jax.experimental import pallas as pl
from jax.experimental.pallas import tpu as pltpu
from jax.experimental.pallas import tpu_sc as plsc

B, S, H = 8, 4096, 1024
CLASS_ID, TEXT_ID = 1, 2
C = 16 + B - 1
CROWS = 24
CPAD = 32
SCW = 48
LANES = 16
NSTRIP = H // LANES
MAX_TEXT = S - (16 * 8 + 2)
CS = 1024
NCHUNK = S // CS
SCC = 2
TCN = NCHUNK - SCC
NW = 32
TBLW = 16
PROWS = 32


def _idx_body(ids_ref, attn_ref, scal_ref, tbl_ref):
    ids = ids_ref[...]
    attn = attn_ref[...]
    pos = lax.broadcasted_iota(jnp.int32, (B, S), 1)
    cmask = ids == CLASS_ID
    ncl = jnp.sum(jnp.where(cmask, 1, 0), axis=1, keepdims=True)
    ts = jnp.min(jnp.where(ids == TEXT_ID, pos, S), axis=1, keepdims=True)
    ts = jnp.where(ts >= S, 0, ts)
    eos = jnp.max(jnp.where(attn != 0, pos, -1), axis=1, keepdims=True)
    eos = jnp.where(eos < 0, S - 1, eos)
    tmask = (attn != 0) & (pos >= ts) & (pos < eos) & (pos < ts + MAX_TEXT)
    cnt = jnp.sum(jnp.where(tmask, 1, 0), axis=1, keepdims=True)
    tcn = jnp.maximum(1, eos // CS + 1 - SCC)
    sc_start = jnp.maximum(ts, tcn * CS)
    sc_end = jnp.minimum(eos, ts + MAX_TEXT)
    prev = jnp.full((B, 1), -1, jnp.int32)
    for c in range(CROWS):
        cur = jnp.min(jnp.where(cmask & (pos > prev), pos, S), axis=1,
                      keepdims=True)
        scal_ref[:, c:c + 1] = jnp.where(cur < S, cur, 0)
        prev = cur
    for c in range(CROWS, 32):
        scal_ref[:, c:c + 1] = jnp.zeros((B, 1), jnp.int32)
    scal_ref[:, 32:33] = ts
    scal_ref[:, 33:34] = eos
    scal_ref[:, 34:35] = ncl
    scal_ref[:, 35:36] = cnt
    scal_ref[:, 36:37] = tcn
    scal_ref[:, 37:38] = sc_start
    scal_ref[:, 38:39] = sc_end
    for c in range(39, SCW):
        scal_ref[:, c:c + 1] = jnp.zeros((B, 1), jnp.int32)

    span = jnp.maximum(sc_end - sc_start, 0)
    bio = lax.broadcasted_iota(jnp.int32, (B, 1), 0)
    tri = (lax.broadcasted_iota(jnp.int32, (B, B), 0)
           > lax.broadcasted_iota(jnp.int32, (B, B), 1))
    cum = jnp.dot(tri.astype(jnp.float32), span.astype(jnp.float32),
                  preferred_element_type=jnp.float32).astype(jnp.int32)
    total = jnp.sum(span)

    def sel(arr, i):
        return jnp.sum(jnp.where(bio == i, arr, 0))

    zero = jnp.zeros((1, 1), jnp.int32)

    def put(w, col, val):
        tbl_ref[w:w + 1, col:col + 1] = jnp.full((1, 1), val, jnp.int32)

    for w in range(NW):
        vs = w * total // NW
        ve = (w + 1) * total // NW
        b0 = jnp.sum(jnp.where(cum <= vs, 1, 0)) - 1
        c0 = sel(cum, b0)
        s0 = sel(sc_start, b0)
        e0 = sel(sc_end, b0)
        r0 = s0 + (vs - c0)
        cross = ve > c0 + sel(span, b0)
        j0b = jnp.where(ve > vs, b0, -1)
        j0e = jnp.where(cross, e0, r0 + (ve - vs))
        b1 = b0 + 1
        s1 = sel(sc_start, b1)
        c1 = sel(cum, b1)
        j1b = jnp.where(cross, b1, -1)
        j1e = jnp.where(cross, s1 + (ve - c1), 0)
        put(w, 0, j0b)
        put(w, 1, r0)
        put(w, 2, j0e)
        put(w, 3, j1b)
        put(w, 4, jnp.where(cross, s1, 0))
        put(w, 5, j1e)
        for col in range(6, TBLW):
            tbl_ref[w:w + 1, col:col + 1] = zero


def _sc_body(scal_hbm, tbl_hbm, emb_hbm, cls_out, part_out,
             hdr_v, idx_v, rows_v, buf0_v, buf1_v, acc_v, semc, sem0, sem1):
    cid = lax.axis_index("c")
    sid = lax.axis_index("s")
    wid = sid * 2 + cid

    @pl.when(wid < B)
    def _():
        pltpu.sync_copy(scal_hbm.at[wid, pl.ds(0, CPAD)], idx_v)
        off = wid * S
        idx_v[pl.ds(0, LANES)] = idx_v[pl.ds(0, LANES)] + off
        idx_v[pl.ds(LANES, LANES)] = idx_v[pl.ds(LANES, LANES)] + off
        pltpu.async_copy(emb_hbm.at[idx_v], rows_v, semc)

    pltpu.sync_copy(tbl_hbm.at[wid], hdr_v)
    tv = hdr_v[pl.ds(0, LANES)]

    def zero_row(buf, r):
        def zr(s, carry):
            buf[r, pl.ds(s * LANES, LANES)] = jnp.zeros((LANES,), jnp.float32)
            return carry
        lax.fori_loop(0, NSTRIP, zr, 0)

    def accum_piece(buf):
        def sbody(s, carry):
            off = s * LANES
            lanes = [buf[r, pl.ds(off, LANES)] for r in range(PROWS)]
            a0 = acc_v[pl.ds(off, LANES)]
            a1, a2, a3 = lanes[1], lanes[2], lanes[3]
            a0 = a0 + lanes[0]
            for r in range(4, PROWS, 4):
                a0 = a0 + lanes[r]
                a1 = a1 + lanes[r + 1]
                a2 = a2 + lanes[r + 2]
                a3 = a3 + lanes[r + 3]
            acc_v[pl.ds(off, LANES)] = (a0 + a1) + (a2 + a3)
            return carry
        lax.fori_loop(0, NSTRIP, sbody, 0)

    def accum8(buf):
        def sbody(s, carry):
            off = s * LANES
            a0 = acc_v[pl.ds(off, LANES)] + buf[0, pl.ds(off, LANES)]
            a1 = buf[1, pl.ds(off, LANES)] + buf[2, pl.ds(off, LANES)]
            a2 = buf[3, pl.ds(off, LANES)] + buf[4, pl.ds(off, LANES)]
            a3 = buf[5, pl.ds(off, LANES)] + buf[6, pl.ds(off, LANES)]
            a0 = a0 + buf[7, pl.ds(off, LANES)]
            acc_v[pl.ds(off, LANES)] = (a0 + a1) + (a2 + a3)
            return carry
        lax.fori_loop(0, NSTRIP, sbody, 0)

    def run_job(bb, v0, v1, slot):
        bb = jnp.maximum(bb, 0)
        my0 = (v0 // 8) * 8
        h = v0 - my0
        has_head = (h > 0) & (v1 > my0)
        core = my0 + jnp.where(has_head, 8, 0)
        nf = jnp.maximum(v1 - core, 0) // PROWS
        base = bb * S + core

        def zbody(s, carry):
            acc_v[pl.ds(s * LANES, LANES)] = jnp.zeros((LANES,), jnp.float32)
            return carry
        lax.fori_loop(0, NSTRIP, zbody, 0)

        @pl.when(has_head)
        def _():
            pltpu.sync_copy(emb_hbm.at[pl.ds(bb * S + my0, 8)],
                            buf0_v.at[pl.ds(0, 8)])

            def _head_fix(r):
                @pl.when((r < h) | (my0 + r >= v1))
                def _():
                    zero_row(buf0_v, r)
            for r in range(8):
                _head_fix(r)
            accum8(buf0_v)

        @pl.when(nf > 0)
        def _():
            pltpu.async_copy(emb_hbm.at[pl.ds(base, PROWS)], buf0_v, sem0)

        @pl.when(nf > 1)
        def _():
            pltpu.async_copy(emb_hbm.at[pl.ds(base + PROWS, PROWS)],
                             buf1_v, sem1)

        def pbody(i, carry):
            p0 = 2 * i

            @pl.when(p0 < nf)
            def _():
                pltpu.make_async_copy(emb_hbm.at[pl.ds(0, PROWS)], buf0_v,
                                      sem0).wait()
                accum_piece(buf0_v)

                @pl.when(p0 + 2 < nf)
                def _():
                    pltpu.async_copy(
                        emb_hbm.at[pl.ds(base + (p0 + 2) * PROWS, PROWS)],
                        buf0_v, sem0)

            @pl.when(p0 + 1 < nf)
            def _():
                pltpu.make_async_copy(emb_hbm.at[pl.ds(0, PROWS)], buf1_v,
                                      sem1).wait()
                accum_piece(buf1_v)

                @pl.when(p0 + 3 < nf)
                def _():
                    pltpu.async_copy(
                        emb_hbm.at[pl.ds(base + (p0 + 3) * PROWS, PROWS)],
                        buf1_v, sem1)

            return carry

        lax.fori_loop(0, (nf + 1) // 2, pbody, 0)

        r0 = core + nf * PROWS
        rem = jnp.maximum(v1 - r0, 0)
        nrem8 = rem // 8
        fr = rem - nrem8 * 8

        def rbody(i, carry):
            pltpu.sync_copy(emb_hbm.at[pl.ds(bb * S + r0 + i * 8, 8)],
                            buf0_v.at[pl.ds(0, 8)])
            accum8(buf0_v)
            return carry
        lax.fori_loop(0, nrem8, rbody, 0)

        @pl.when(fr > 0)
        def _():
            pltpu.sync_copy(emb_hbm.at[pl.ds(bb * S + r0 + nrem8 * 8, 8)],
                            buf0_v.at[pl.ds(0, 8)])

            def _tail_fix(r):
                @pl.when(r >= fr)
                def _():
                    zero_row(buf0_v, r)
            for r in range(1, 8):
                _tail_fix(r)
            accum8(buf0_v)

        pltpu.sync_copy(acc_v, part_out.at[wid, slot])

    run_job(tv[0], tv[1], tv[2], 0)
    run_job(tv[3], tv[4], tv[5], 1)

    @pl.when(wid < B)
    def _():
        pltpu.make_async_copy(emb_hbm.at[idx_v], rows_v, semc).wait()
        pltpu.sync_copy(rows_v.at[pl.ds(0, CROWS)], cls_out.at[wid])


def _make_sc_call():
    return pl.kernel(
        _sc_body,
        out_type=(jax.ShapeDtypeStruct((B, CROWS, H), jnp.float32),
                  jax.ShapeDtypeStruct((NW, 2, H), jnp.float32)),
        mesh=plsc.VectorSubcoreMesh(core_axis_name="c", subcore_axis_name="s"),
        scratch_types=[
            pltpu.VMEM((TBLW,), jnp.int32),
            pltpu.VMEM((CPAD,), jnp.int32),
            pltpu.VMEM((CPAD, H), jnp.float32),
            pltpu.VMEM((PROWS, H), jnp.float32),
            pltpu.VMEM((PROWS, H), jnp.float32),
            pltpu.VMEM((H,), jnp.float32),
            pltpu.SemaphoreType.DMA,
            pltpu.SemaphoreType.DMA,
            pltpu.SemaphoreType.DMA,
        ],
    )


def _tc_body(scal_ref, emb_ref, attn_ref, acc_ref, acc):
    b = pl.program_id(0)
    k = pl.program_id(1)
    ts = scal_ref[b, 32]
    eos = scal_ref[b, 33]
    tcn = scal_ref[b, 36]

    @pl.when(k == 0)
    def _():
        acc[...] = jnp.zeros_like(acc)

    @pl.when(k < tcn)
    def _():
        posr = k * CS + lax.broadcasted_iota(jnp.int32, (1, CS), 1)
        att = attn_ref[0, 0, pl.ds(k * CS, CS)]
        m = ((posr >= ts) & (posr < eos) & (posr < ts + MAX_TEXT)
             & (att[None, :] != 0))
        mf = m.astype(jnp.float32)
        chunk = emb_ref[...].reshape(CS, H)
        acc[...] += jnp.dot(mf, chunk, preferred_element_type=jnp.float32)

    @pl.when(k == TCN - 1)
    def _():
        acc_ref[pl.ds(b, 1), :] = acc[...]


def _final_body(acc_ref, part_ref, tbl_ref, cls_ref, wt_ref, wc_ref,
                scal_ref, scale_ref, out_ref):
    scale = scale_ref[0, 0]
    t = tbl_ref[...]
    wio = lax.broadcasted_iota(jnp.int32, (NW, B), 1)
    sel0 = (t[:, 0:1] == wio).astype(jnp.float32)
    sel1 = (t[:, 3:4] == wio).astype(jnp.float32)
    psum = (lax.dot_general(sel0, part_ref[:, 0, :],
                            (((0,), (0,)), ((), ())),
                            preferred_element_type=jnp.float32)
            + lax.dot_general(sel1, part_ref[:, 1, :],
                              (((0,), (0,)), ((), ())),
                              preferred_element_type=jnp.float32))
    for b in range(B):
        cnt = scal_ref[b, 35].astype(jnp.float32)
        pooled = (acc_ref[b:b + 1, :] + psum[b:b + 1, :]) / (cnt + 1e-8)
        text_rep = jnp.dot(pooled, wt_ref[...],
                           preferred_element_type=jnp.float32)
        u = lax.dot_general(text_rep, wc_ref[...],
                            (((1,), (1,)), ((), ())),
                            preferred_element_type=jnp.float32)
        lo = lax.dot_general(u, cls_ref[b], (((1,), (1,)), ((), ())),
                             preferred_element_type=jnp.float32)
        cio = lax.broadcasted_iota(jnp.int32, (1, CROWS), 1)
        lo = jnp.where(cio < scal_ref[b, 34], lo, 0.0) * scale
        pad = jnp.zeros((1, 128 - CROWS), jnp.float32)
        out_ref[b:b + 1, :] = jnp.concatenate([lo, pad], axis=1)


def kernel(token_embeds, input_ids, attention_mask, W_text, W_class,
           logit_scale):
    ids = input_ids.astype(jnp.int32)
    attn = attention_mask.astype(jnp.int32)
    emb_flat = token_embeds.reshape(B * S, H)

    scal, tbl = pl.pallas_call(
        _idx_body,
        out_shape=(jax.ShapeDtypeStruct((B, SCW), jnp.int32),
                   jax.ShapeDtypeStruct((NW, TBLW), jnp.int32)),
    )(ids, attn)

    cls_rows, partials = _make_sc_call()(scal, tbl, emb_flat)

    attn3 = attn.reshape(B, 1, S)
    scale2d = logit_scale.astype(jnp.float32).reshape(1, 1)

    grid_spec = pltpu.PrefetchScalarGridSpec(
        num_scalar_prefetch=1,
        grid=(B, TCN),
        in_specs=[
            pl.BlockSpec((1, CS, H),
                         lambda b, k, sc: (b, jnp.minimum(k, sc[b, 36] - 1), 0)),
            pl.BlockSpec((1, 1, S), lambda b, k, sc: (b, 0, 0)),
        ],
        out_specs=pl.BlockSpec((8, H), lambda b, k, sc: (0, 0)),
        scratch_shapes=[
            pltpu.VMEM((1, H), jnp.float32),
        ],
    )
    acc = pl.pallas_call(
        _tc_body,
        grid_spec=grid_spec,
        out_shape=jax.ShapeDtypeStruct((8, H), jnp.float32),
        compiler_params=pltpu.CompilerParams(
            dimension_semantics=("arbitrary", "arbitrary")),
    )(scal, token_embeds, attn3)

    out = pl.pallas_call(
        _final_body,
        in_specs=[
            pl.BlockSpec((8, H), lambda: (0, 0)),
            pl.BlockSpec((NW, 2, H), lambda: (0, 0, 0)),
            pl.BlockSpec((NW, TBLW), lambda: (0, 0)),
            pl.BlockSpec((B, CROWS, H), lambda: (0, 0, 0)),
            pl.BlockSpec((H, H), lambda: (0, 0)),
            pl.BlockSpec((H, H), lambda: (0, 0)),
            pl.BlockSpec(memory_space=pltpu.SMEM),
            pl.BlockSpec(memory_space=pltpu.SMEM),
        ],
        out_shape=jax.ShapeDtypeStruct((8, 128), jnp.float32),
    )(acc, partials, tbl, cls_rows, W_text, W_class, scal, scale2d)
    return out[:B, :C]


# speedup vs baseline: 1.1083x; 1.1083x over previous
import jax
import jax.numpy as jnp
from jax import lax
from jax.experimental import pallas as pl
from jax.experimental.pallas import tpu as pltpu
from jax.experimental.pallas import tpu_sc as plsc

B, S, H = 8, 4096, 1024
CLASS_ID, TEXT_ID = 1, 2
C = 16 + B - 1
CROWS = 24
CPAD = 32
SCW = 48
LANES = 16
NSTRIP = H // LANES
MAX_TEXT = S - (16 * 8 + 2)
CS = 512
NCHUNK = S // CS
SCC = 3
TCN = NCHUNK - SCC
NW = 32
TBLW = 16
PROWS = 32


def _idx_body(ids_ref, attn_ref, scal_ref, tbl_ref):
    ids = ids_ref[...]
    attn = attn_ref[...]
    pos = lax.broadcasted_iota(jnp.int32, (B, S), 1)
    cmask = ids == CLASS_ID
    ncl = jnp.sum(jnp.where(cmask, 1, 0), axis=1, keepdims=True)
    ts = jnp.min(jnp.where(ids == TEXT_ID, pos, S), axis=1, keepdims=True)
    ts = jnp.where(ts >= S, 0, ts)
    eos = jnp.max(jnp.where(attn != 0, pos, -1), axis=1, keepdims=True)
    eos = jnp.where(eos < 0, S - 1, eos)
    tmask = (attn != 0) & (pos >= ts) & (pos < eos) & (pos < ts + MAX_TEXT)
    cnt = jnp.sum(jnp.where(tmask, 1, 0), axis=1, keepdims=True)
    tcn = jnp.maximum(1, eos // CS + 1 - SCC)
    sc_start = jnp.maximum(ts, tcn * CS)
    sc_end = jnp.minimum(eos, ts + MAX_TEXT)
    prev = jnp.full((B, 1), -1, jnp.int32)
    for c in range(CROWS):
        cur = jnp.min(jnp.where(cmask & (pos > prev), pos, S), axis=1,
                      keepdims=True)
        scal_ref[:, c:c + 1] = jnp.where(cur < S, cur, 0)
        prev = cur
    for c in range(CROWS, 32):
        scal_ref[:, c:c + 1] = jnp.zeros((B, 1), jnp.int32)
    scal_ref[:, 32:33] = ts
    scal_ref[:, 33:34] = eos
    scal_ref[:, 34:35] = ncl
    scal_ref[:, 35:36] = cnt
    scal_ref[:, 36:37] = tcn
    scal_ref[:, 37:38] = sc_start
    scal_ref[:, 38:39] = sc_end
    for c in range(39, SCW):
        scal_ref[:, c:c + 1] = jnp.zeros((B, 1), jnp.int32)

    bio = lax.broadcasted_iota(jnp.int32, (B, 1), 0)
    sa8 = (sc_start // 8) * 8
    span8 = jnp.maximum(sc_end - sa8, 0)
    tot8 = jnp.maximum(jnp.sum(span8), 1)
    counts = 1 + span8 * (NW - B) // tot8
    left = NW - jnp.sum(counts)
    counts = counts + jnp.where(bio < left, 1, 0)
    tri = (lax.broadcasted_iota(jnp.int32, (B, B), 0)
           > lax.broadcasted_iota(jnp.int32, (B, B), 1))
    ccum = jnp.dot(tri.astype(jnp.float32), counts.astype(jnp.float32),
                   preferred_element_type=jnp.float32).astype(jnp.int32)

    def sel(arr, i):
        return jnp.sum(jnp.where(bio == i, arr, 0))

    zero = jnp.zeros((1, 1), jnp.int32)

    def put(w, col, val):
        tbl_ref[w:w + 1, col:col + 1] = jnp.full((1, 1), val, jnp.int32)

    for w in range(NW):
        b0 = jnp.sum(jnp.where(ccum <= w, 1, 0)) - 1
        j = w - sel(ccum, b0)
        nb = sel(counts, b0)
        sa0 = sel(sa8, b0)
        s0 = sel(sc_start, b0)
        e0 = sel(sc_end, b0)
        sp = jnp.maximum(e0 - sa0, 0)
        q8 = (((sp + nb - 1) // nb + 7) // 8) * 8
        v0 = sa0 + j * q8
        v1 = jnp.minimum(v0 + q8, e0)
        v0c = jnp.maximum(v0, s0)
        put(w, 0, jnp.where(v1 > v0c, b0, -1))
        put(w, 1, v0c)
        put(w, 2, v1)
        put(w, 3, jnp.full((), -1, jnp.int32))
        for col in range(4, TBLW):
            tbl_ref[w:w + 1, col:col + 1] = zero


def _sc_body(scal_hbm, tbl_hbm, emb_hbm, cls_out, part_out,
             hdr_v, idx_v, rows_v, buf0_v, buf1_v, acc_v, semc, sem0, sem1):
    cid = lax.axis_index("c")
    sid = lax.axis_index("s")
    wid = sid * 2 + cid

    @pl.when(wid < B)
    def _():
        pltpu.sync_copy(scal_hbm.at[wid, pl.ds(0, CPAD)], idx_v)
        off = wid * S
        idx_v[pl.ds(0, LANES)] = idx_v[pl.ds(0, LANES)] + off
        idx_v[pl.ds(LANES, LANES)] = idx_v[pl.ds(LANES, LANES)] + off
        pltpu.async_copy(emb_hbm.at[idx_v], rows_v, semc)

    pltpu.sync_copy(tbl_hbm.at[wid], hdr_v)
    tv = hdr_v[pl.ds(0, LANES)]

    def zero_row(buf, r):
        def zr(s, carry):
            buf[r, pl.ds(s * LANES, LANES)] = jnp.zeros((LANES,), jnp.float32)
            return carry
        lax.fori_loop(0, NSTRIP, zr, 0)

    def accum_piece(buf):
        def sbody(s, carry):
            off = s * LANES
            lanes = [buf[r, pl.ds(off, LANES)] for r in range(PROWS)]
            a0 = acc_v[pl.ds(off, LANES)]
            a1, a2, a3 = lanes[1], lanes[2], lanes[3]
            a0 = a0 + lanes[0]
            for r in range(4, PROWS, 4):
                a0 = a0 + lanes[r]
                a1 = a1 + lanes[r + 1]
                a2 = a2 + lanes[r + 2]
                a3 = a3 + lanes[r + 3]
            acc_v[pl.ds(off, LANES)] = (a0 + a1) + (a2 + a3)
            return carry
        lax.fori_loop(0, NSTRIP, sbody, 0)

    def accum8(buf):
        def sbody(s, carry):
            off = s * LANES
            a0 = acc_v[pl.ds(off, LANES)] + buf[0, pl.ds(off, LANES)]
            a1 = buf[1, pl.ds(off, LANES)] + buf[2, pl.ds(off, LANES)]
            a2 = buf[3, pl.ds(off, LANES)] + buf[4, pl.ds(off, LANES)]
            a3 = buf[5, pl.ds(off, LANES)] + buf[6, pl.ds(off, LANES)]
            a0 = a0 + buf[7, pl.ds(off, LANES)]
            acc_v[pl.ds(off, LANES)] = (a0 + a1) + (a2 + a3)
            return carry
        lax.fori_loop(0, NSTRIP, sbody, 0)

    def run_job(bb, v0, v1, slot):
        bb = jnp.maximum(bb, 0)
        my0 = (v0 // 8) * 8
        h = v0 - my0
        has_head = (h > 0) & (v1 > my0)
        core = my0 + jnp.where(has_head, 8, 0)
        nf = jnp.maximum(v1 - core, 0) // PROWS
        base = bb * S + core

        def zbody(s, carry):
            acc_v[pl.ds(s * LANES, LANES)] = jnp.zeros((LANES,), jnp.float32)
            return carry
        lax.fori_loop(0, NSTRIP, zbody, 0)

        @pl.when(has_head)
        def _():
            pltpu.sync_copy(emb_hbm.at[pl.ds(bb * S + my0, 8)],
                            buf0_v.at[pl.ds(0, 8)])

            def _head_fix(r):
                @pl.when((r < h) | (my0 + r >= v1))
                def _():
                    zero_row(buf0_v, r)
            for r in range(8):
                _head_fix(r)
            accum8(buf0_v)

        @pl.when(nf > 0)
        def _():
            pltpu.async_copy(emb_hbm.at[pl.ds(base, PROWS)], buf0_v, sem0)

        @pl.when(nf > 1)
        def _():
            pltpu.async_copy(emb_hbm.at[pl.ds(base + PROWS, PROWS)],
                             buf1_v, sem1)

        def pbody(i, carry):
            p0 = 2 * i

            @pl.when(p0 < nf)
            def _():
                pltpu.make_async_copy(emb_hbm.at[pl.ds(0, PROWS)], buf0_v,
                                      sem0).wait()
                accum_piece(buf0_v)

                @pl.when(p0 + 2 < nf)
                def _():
                    pltpu.async_copy(
                        emb_hbm.at[pl.ds(base + (p0 + 2) * PROWS, PROWS)],
                        buf0_v, sem0)

            @pl.when(p0 + 1 < nf)
            def _():
                pltpu.make_async_copy(emb_hbm.at[pl.ds(0, PROWS)], buf1_v,
                                      sem1).wait()
                accum_piece(buf1_v)

                @pl.when(p0 + 3 < nf)
                def _():
                    pltpu.async_copy(
                        emb_hbm.at[pl.ds(base + (p0 + 3) * PROWS, PROWS)],
                        buf1_v, sem1)

            return carry

        lax.fori_loop(0, (nf + 1) // 2, pbody, 0)

        r0 = core + nf * PROWS
        rem = jnp.maximum(v1 - r0, 0)
        nrem8 = rem // 8
        fr = rem - nrem8 * 8

        def rbody(i, carry):
            pltpu.sync_copy(emb_hbm.at[pl.ds(bb * S + r0 + i * 8, 8)],
                            buf0_v.at[pl.ds(0, 8)])
            accum8(buf0_v)
            return carry
        lax.fori_loop(0, nrem8, rbody, 0)

        @pl.when(fr > 0)
        def _():
            pltpu.sync_copy(emb_hbm.at[pl.ds(bb * S + r0 + nrem8 * 8, 8)],
                            buf0_v.at[pl.ds(0, 8)])

            def _tail_fix(r):
                @pl.when(r >= fr)
                def _():
                    zero_row(buf0_v, r)
            for r in range(1, 8):
                _tail_fix(r)
            accum8(buf0_v)

        pltpu.sync_copy(acc_v, part_out.at[wid, slot])

    run_job(tv[0], tv[1], tv[2], 0)
    run_job(tv[3], tv[4], tv[5], 1)

    @pl.when(wid < B)
    def _():
        pltpu.make_async_copy(emb_hbm.at[idx_v], rows_v, semc).wait()
        pltpu.sync_copy(rows_v.at[pl.ds(0, CROWS)], cls_out.at[wid])


def _make_sc_call():
    return pl.kernel(
        _sc_body,
        out_type=(jax.ShapeDtypeStruct((B, CROWS, H), jnp.float32),
                  jax.ShapeDtypeStruct((NW, 2, H), jnp.float32)),
        mesh=plsc.VectorSubcoreMesh(core_axis_name="c", subcore_axis_name="s"),
        scratch_types=[
            pltpu.VMEM((TBLW,), jnp.int32),
            pltpu.VMEM((CPAD,), jnp.int32),
            pltpu.VMEM((CPAD, H), jnp.float32),
            pltpu.VMEM((PROWS, H), jnp.float32),
            pltpu.VMEM((PROWS, H), jnp.float32),
            pltpu.VMEM((H,), jnp.float32),
            pltpu.SemaphoreType.DMA,
            pltpu.SemaphoreType.DMA,
            pltpu.SemaphoreType.DMA,
        ],
    )


def _tc_body(scal_ref, emb_ref, attn_ref, acc_ref, acc):
    b = pl.program_id(0)
    k = pl.program_id(1)
    ts = scal_ref[b, 32]
    eos = scal_ref[b, 33]
    tcn = scal_ref[b, 36]

    @pl.when(k == 0)
    def _():
        acc[...] = jnp.zeros_like(acc)

    @pl.when(k < tcn)
    def _():
        posr = k * CS + lax.broadcasted_iota(jnp.int32, (1, CS), 1)
        att = attn_ref[0, 0, pl.ds(k * CS, CS)]
        m = ((posr >= ts) & (posr < eos) & (posr < ts + MAX_TEXT)
             & (att[None, :] != 0))
        mf = m.astype(jnp.float32)
        chunk = emb_ref[...].reshape(CS, H)
        acc[...] += jnp.dot(mf, chunk, preferred_element_type=jnp.float32)

    @pl.when(k == TCN - 1)
    def _():
        acc_ref[pl.ds(b, 1), :] = acc[...]


def _final_body(acc_ref, part_ref, tbl_ref, cls_ref, wt_ref, wc_ref,
                scal_ref, scale_ref, out_ref):
    scale = scale_ref[0, 0]
    t = tbl_ref[...]
    wio = lax.broadcasted_iota(jnp.int32, (NW, B), 1)
    sel0 = (t[:, 0:1] == wio).astype(jnp.float32)
    sel1 = (t[:, 3:4] == wio).astype(jnp.float32)
    psum = (lax.dot_general(sel0, part_ref[:, 0, :],
                            (((0,), (0,)), ((), ())),
                            preferred_element_type=jnp.float32)
            + lax.dot_general(sel1, part_ref[:, 1, :],
                              (((0,), (0,)), ((), ())),
                              preferred_element_type=jnp.float32))
    for b in range(B):
        cnt = scal_ref[b, 35].astype(jnp.float32)
        pooled = (acc_ref[b:b + 1, :] + psum[b:b + 1, :]) / (cnt + 1e-8)
        text_rep = jnp.dot(pooled, wt_ref[...],
                           preferred_element_type=jnp.float32)
        u = lax.dot_general(text_rep, wc_ref[...],
                            (((1,), (1,)), ((), ())),
                            preferred_element_type=jnp.float32)
        lo = lax.dot_general(u, cls_ref[b], (((1,), (1,)), ((), ())),
                             preferred_element_type=jnp.float32)
        cio = lax.broadcasted_iota(jnp.int32, (1, CROWS), 1)
        lo = jnp.where(cio < scal_ref[b, 34], lo, 0.0) * scale
        pad = jnp.zeros((1, 128 - CROWS), jnp.float32)
        out_ref[b:b + 1, :] = jnp.concatenate([lo, pad], axis=1)


def kernel(token_embeds, input_ids, attention_mask, W_text, W_class,
           logit_scale):
    ids = input_ids.astype(jnp.int32)
    attn = attention_mask.astype(jnp.int32)
    emb_flat = token_embeds.reshape(B * S, H)

    scal, tbl = pl.pallas_call(
        _idx_body,
        out_shape=(jax.ShapeDtypeStruct((B, SCW), jnp.int32),
                   jax.ShapeDtypeStruct((NW, TBLW), jnp.int32)),
    )(ids, attn)

    cls_rows, partials = _make_sc_call()(scal, tbl, emb_flat)

    attn3 = attn.reshape(B, 1, S)
    scale2d = logit_scale.astype(jnp.float32).reshape(1, 1)

    grid_spec = pltpu.PrefetchScalarGridSpec(
        num_scalar_prefetch=1,
        grid=(B, TCN),
        in_specs=[
            pl.BlockSpec((1, CS, H),
                         lambda b, k, sc: (b, jnp.minimum(k, sc[b, 36] - 1), 0)),
            pl.BlockSpec((1, 1, S), lambda b, k, sc: (b, 0, 0)),
        ],
        out_specs=pl.BlockSpec((8, H), lambda b, k, sc: (0, 0)),
        scratch_shapes=[
            pltpu.VMEM((1, H), jnp.float32),
        ],
    )
    acc = pl.pallas_call(
        _tc_body,
        grid_spec=grid_spec,
        out_shape=jax.ShapeDtypeStruct((8, H), jnp.float32),
        compiler_params=pltpu.CompilerParams(
            dimension_semantics=("arbitrary", "arbitrary")),
    )(scal, token_embeds, attn3)

    out = pl.pallas_call(
        _final_body,
        in_specs=[
            pl.BlockSpec((8, H), lambda: (0, 0)),
            pl.BlockSpec((NW, 2, H), lambda: (0, 0, 0)),
            pl.BlockSpec((NW, TBLW), lambda: (0, 0)),
            pl.BlockSpec((B, CROWS, H), lambda: (0, 0, 0)),
            pl.BlockSpec((H, H), lambda: (0, 0)),
            pl.BlockSpec((H, H), lambda: (0, 0)),
            pl.BlockSpec(memory_space=pltpu.SMEM),
            pl.BlockSpec(memory_space=pltpu.SMEM),
        ],
        out_shape=jax.ShapeDtypeStruct((8, 128), jnp.float32),
    )(acc, partials, tbl, cls_rows, W_text, W_class, scal, scale2d)
    return out[:B, :C]


# speedup vs baseline: 1.1473x; 1.0352x over previous
import jax
import jax.numpy as jnp
from jax import lax
from jax.experimental import pallas as pl
from jax.experimental.pallas import tpu as pltpu
from jax.experimental.pallas import tpu_sc as plsc

B, S, H = 8, 4096, 1024
CLASS_ID, TEXT_ID = 1, 2
C = 16 + B - 1
CROWS = 24
CPAD = 32
SCW = 48
LANES = 16
NSTRIP = H // LANES
MAX_TEXT = S - (16 * 8 + 2)
CS = 1024
NCHUNK = S // CS
SCC = 1
TCN = NCHUNK - SCC
NQ = 4
PROWS = 32


def _idx_body(ids_ref, attn_ref, scal_ref):
    ids = ids_ref[...]
    attn = attn_ref[...]
    pos = lax.broadcasted_iota(jnp.int32, (B, S), 1)
    cmask = ids == CLASS_ID
    ncl = jnp.sum(jnp.where(cmask, 1, 0), axis=1, keepdims=True)
    ts = jnp.min(jnp.where(ids == TEXT_ID, pos, S), axis=1, keepdims=True)
    ts = jnp.where(ts >= S, 0, ts)
    eos = jnp.max(jnp.where(attn != 0, pos, -1), axis=1, keepdims=True)
    eos = jnp.where(eos < 0, S - 1, eos)
    tmask = (attn != 0) & (pos >= ts) & (pos < eos) & (pos < ts + MAX_TEXT)
    cnt = jnp.sum(jnp.where(tmask, 1, 0), axis=1, keepdims=True)
    tcn = jnp.maximum(1, eos // CS + 1 - SCC)
    sc_start = jnp.maximum(ts, tcn * CS)
    sc_end = jnp.minimum(eos, ts + MAX_TEXT)
    prev = jnp.full((B, 1), -1, jnp.int32)
    for c in range(CROWS):
        cur = jnp.min(jnp.where(cmask & (pos > prev), pos, S), axis=1,
                      keepdims=True)
        scal_ref[:, c:c + 1] = jnp.where(cur < S, cur, 0)
        prev = cur
    for c in range(CROWS, 32):
        scal_ref[:, c:c + 1] = jnp.zeros((B, 1), jnp.int32)
    scal_ref[:, 32:33] = ts
    scal_ref[:, 33:34] = eos
    scal_ref[:, 34:35] = ncl
    scal_ref[:, 35:36] = cnt
    scal_ref[:, 36:37] = tcn
    scal_ref[:, 37:38] = sc_start
    scal_ref[:, 38:39] = sc_end
    for c in range(39, SCW):
        scal_ref[:, c:c + 1] = jnp.zeros((B, 1), jnp.int32)


def _sc_body(scal_hbm, emb_hbm, cls_out, part_out,
             hdr_v, idx_v, rows_v, buf0_v, buf1_v, acc_v, semc, sem0, sem1):
    cid = lax.axis_index("c")
    sid = lax.axis_index("s")
    wid = sid * 2 + cid
    b = wid // NQ
    j = wid - b * NQ

    pltpu.sync_copy(scal_hbm.at[b], hdr_v)
    hv = hdr_v[pl.ds(32, LANES)]
    sc_start = hv[5]
    sc_end = hv[6]
    sa = (sc_start // 8) * 8
    length = jnp.maximum(sc_end - sa, 0)
    q = (((length + (NQ - 1)) // NQ + 7) // 8) * 8
    my0 = sa + j * q
    v0 = jnp.maximum(my0, sc_start)
    v1 = jnp.minimum(jnp.minimum(my0 + q, sc_end), S)
    h = v0 - my0
    has_head = (h > 0) & (v1 > my0)
    core = my0 + jnp.where(has_head, 8, 0)
    nf = jnp.maximum(v1 - core, 0) // PROWS
    base = b * S + core

    @pl.when(j == 0)
    def _():
        pltpu.sync_copy(scal_hbm.at[b, pl.ds(0, CPAD)], idx_v)
        off = b * S
        idx_v[pl.ds(0, LANES)] = idx_v[pl.ds(0, LANES)] + off
        idx_v[pl.ds(LANES, LANES)] = idx_v[pl.ds(LANES, LANES)] + off
        pltpu.async_copy(emb_hbm.at[idx_v], rows_v, semc)

    def zbody(s, carry):
        acc_v[pl.ds(s * LANES, LANES)] = jnp.zeros((LANES,), jnp.float32)
        return carry
    lax.fori_loop(0, NSTRIP, zbody, 0)

    def zero_row(buf, r):
        def zr(s, carry):
            buf[r, pl.ds(s * LANES, LANES)] = jnp.zeros((LANES,), jnp.float32)
            return carry
        lax.fori_loop(0, NSTRIP, zr, 0)

    def accum8(buf):
        def sbody(s, carry):
            off = s * LANES
            a0 = acc_v[pl.ds(off, LANES)] + buf[0, pl.ds(off, LANES)]
            a1 = buf[1, pl.ds(off, LANES)] + buf[2, pl.ds(off, LANES)]
            a2 = buf[3, pl.ds(off, LANES)] + buf[4, pl.ds(off, LANES)]
            a3 = buf[5, pl.ds(off, LANES)] + buf[6, pl.ds(off, LANES)]
            a0 = a0 + buf[7, pl.ds(off, LANES)]
            acc_v[pl.ds(off, LANES)] = (a0 + a1) + (a2 + a3)
            return carry
        lax.fori_loop(0, NSTRIP, sbody, 0)

    @pl.when(has_head)
    def _():
        pltpu.sync_copy(emb_hbm.at[pl.ds(b * S + my0, 8)],
                        buf0_v.at[pl.ds(0, 8)])

        def _head_fix(r):
            @pl.when((r < h) | (my0 + r >= v1))
            def _():
                zero_row(buf0_v, r)
        for r in range(8):
            _head_fix(r)
        accum8(buf0_v)

    @pl.when(nf > 0)
    def _():
        pltpu.async_copy(emb_hbm.at[pl.ds(base, PROWS)], buf0_v, sem0)

    @pl.when(nf > 1)
    def _():
        pltpu.async_copy(emb_hbm.at[pl.ds(base + PROWS, PROWS)], buf1_v, sem1)

    def accum_piece(buf):
        def sbody(s, carry):
            off = s * LANES
            lanes = [buf[r, pl.ds(off, LANES)] for r in range(PROWS)]
            a0 = acc_v[pl.ds(off, LANES)]
            a1, a2, a3 = lanes[1], lanes[2], lanes[3]
            a0 = a0 + lanes[0]
            for r in range(4, PROWS, 4):
                a0 = a0 + lanes[r]
                a1 = a1 + lanes[r + 1]
                a2 = a2 + lanes[r + 2]
                a3 = a3 + lanes[r + 3]
            acc_v[pl.ds(off, LANES)] = (a0 + a1) + (a2 + a3)
            return carry
        lax.fori_loop(0, NSTRIP, sbody, 0)

    def pbody(i, carry):
        p0 = 2 * i

        @pl.when(p0 < nf)
        def _():
            pltpu.make_async_copy(emb_hbm.at[pl.ds(0, PROWS)], buf0_v,
                                  sem0).wait()
            accum_piece(buf0_v)

            @pl.when(p0 + 2 < nf)
            def _():
                pltpu.async_copy(
                    emb_hbm.at[pl.ds(base + (p0 + 2) * PROWS, PROWS)],
                    buf0_v, sem0)

        @pl.when(p0 + 1 < nf)
        def _():
            pltpu.make_async_copy(emb_hbm.at[pl.ds(0, PROWS)], buf1_v,
                                  sem1).wait()
            accum_piece(buf1_v)

            @pl.when(p0 + 3 < nf)
            def _():
                pltpu.async_copy(
                    emb_hbm.at[pl.ds(base + (p0 + 3) * PROWS, PROWS)],
                    buf1_v, sem1)

        return carry

    lax.fori_loop(0, (nf + 1) // 2, pbody, 0)

    r0 = core + nf * PROWS
    rem = jnp.maximum(v1 - r0, 0)
    nrem8 = rem // 8
    fr = rem - nrem8 * 8

    def rbody(i, carry):
        pltpu.sync_copy(emb_hbm.at[pl.ds(b * S + r0 + i * 8, 8)],
                        buf0_v.at[pl.ds(0, 8)])
        accum8(buf0_v)
        return carry
    lax.fori_loop(0, nrem8, rbody, 0)

    @pl.when(fr > 0)
    def _():
        pltpu.sync_copy(emb_hbm.at[pl.ds(b * S + r0 + nrem8 * 8, 8)],
                        buf0_v.at[pl.ds(0, 8)])

        def _tail_fix(r):
            @pl.when(r >= fr)
            def _():
                zero_row(buf0_v, r)
        for r in range(1, 8):
            _tail_fix(r)
        accum8(buf0_v)

    pltpu.sync_copy(acc_v, part_out.at[b, j])

    @pl.when(j == 0)
    def _():
        pltpu.make_async_copy(emb_hbm.at[idx_v], rows_v, semc).wait()
        pltpu.sync_copy(rows_v.at[pl.ds(0, CROWS)], cls_out.at[b])


def _make_sc_call():
    return pl.kernel(
        _sc_body,
        out_type=(jax.ShapeDtypeStruct((B, CROWS, H), jnp.float32),
                  jax.ShapeDtypeStruct((B, NQ, H), jnp.float32)),
        mesh=plsc.VectorSubcoreMesh(core_axis_name="c", subcore_axis_name="s"),
        scratch_types=[
            pltpu.VMEM((SCW,), jnp.int32),
            pltpu.VMEM((CPAD,), jnp.int32),
            pltpu.VMEM((CPAD, H), jnp.float32),
            pltpu.VMEM((PROWS, H), jnp.float32),
            pltpu.VMEM((PROWS, H), jnp.float32),
            pltpu.VMEM((H,), jnp.float32),
            pltpu.SemaphoreType.DMA,
            pltpu.SemaphoreType.DMA,
            pltpu.SemaphoreType.DMA,
        ],
    )


def _tc_body(scal_ref, emb_ref, attn_ref, acc_ref, acc):
    b = pl.program_id(0)
    k = pl.program_id(1)
    ts = scal_ref[b, 32]
    eos = scal_ref[b, 33]
    tcn = scal_ref[b, 36]

    @pl.when(k == 0)
    def _():
        acc[...] = jnp.zeros_like(acc)

    @pl.when(k < tcn)
    def _():
        posr = k * CS + lax.broadcasted_iota(jnp.int32, (1, CS), 1)
        att = attn_ref[0, 0, pl.ds(k * CS, CS)]
        m = ((posr >= ts) & (posr < eos) & (posr < ts + MAX_TEXT)
             & (att[None, :] != 0))
        mf = m.astype(jnp.float32)
        chunk = emb_ref[...].reshape(CS, H)
        acc[...] += jnp.dot(mf, chunk, preferred_element_type=jnp.float32)

    @pl.when(k == TCN - 1)
    def _():
        acc_ref[pl.ds(b, 1), :] = acc[...]


def _final_body(acc_ref, part_ref, cls_ref, wt_ref, wc_ref, scal_ref,
                scale_ref, out_ref):
    scale = scale_ref[0, 0]
    for b in range(B):
        psum = jnp.sum(part_ref[b], axis=0, keepdims=True)
        cnt = scal_ref[b, 35].astype(jnp.float32)
        pooled = (acc_ref[b:b + 1, :] + psum) / (cnt + 1e-8)
        text_rep = jnp.dot(pooled, wt_ref[...],
                           preferred_element_type=jnp.float32)
        u = lax.dot_general(text_rep, wc_ref[...],
                            (((1,), (1,)), ((), ())),
                            preferred_element_type=jnp.float32)
        lo = lax.dot_general(u, cls_ref[b], (((1,), (1,)), ((), ())),
                             preferred_element_type=jnp.float32)
        cio = lax.broadcasted_iota(jnp.int32, (1, CROWS), 1)
        lo = jnp.where(cio < scal_ref[b, 34], lo, 0.0) * scale
        pad = jnp.zeros((1, 128 - CROWS), jnp.float32)
        out_ref[b:b + 1, :] = jnp.concatenate([lo, pad], axis=1)


def kernel(token_embeds, input_ids, attention_mask, W_text, W_class,
           logit_scale):
    ids = input_ids.astype(jnp.int32)
    attn = attention_mask.astype(jnp.int32)
    emb_flat = token_embeds.reshape(B * S, H)

    scal = pl.pallas_call(
        _idx_body,
        out_shape=jax.ShapeDtypeStruct((B, SCW), jnp.int32),
    )(ids, attn)

    cls_rows, partials = _make_sc_call()(scal, emb_flat)

    attn3 = attn.reshape(B, 1, S)
    scale2d = logit_scale.astype(jnp.float32).reshape(1, 1)

    grid_spec = pltpu.PrefetchScalarGridSpec(
        num_scalar_prefetch=1,
        grid=(B, TCN),
        in_specs=[
            pl.BlockSpec((1, CS, H),
                         lambda b, k, sc: (b, jnp.minimum(k, sc[b, 36] - 1), 0)),
            pl.BlockSpec((1, 1, S), lambda b, k, sc: (b, 0, 0)),
        ],
        out_specs=pl.BlockSpec((8, H), lambda b, k, sc: (0, 0)),
        scratch_shapes=[
            pltpu.VMEM((1, H), jnp.float32),
        ],
    )
    acc = pl.pallas_call(
        _tc_body,
        grid_spec=grid_spec,
        out_shape=jax.ShapeDtypeStruct((8, H), jnp.float32),
        compiler_params=pltpu.CompilerParams(
            dimension_semantics=("arbitrary", "arbitrary")),
    )(scal, token_embeds, attn3)

    out = pl.pallas_call(
        _final_body,
        in_specs=[
            pl.BlockSpec((8, H), lambda: (0, 0)),
            pl.BlockSpec((B, NQ, H), lambda: (0, 0, 0)),
            pl.BlockSpec((B, CROWS, H), lambda: (0, 0, 0)),
            pl.BlockSpec((H, H), lambda: (0, 0)),
            pl.BlockSpec((H, H), lambda: (0, 0)),
            pl.BlockSpec(memory_space=pltpu.SMEM),
            pl.BlockSpec(memory_space=pltpu.SMEM),
        ],
        out_shape=jax.ShapeDtypeStruct((8, 128), jnp.float32),
    )(acc, partials, cls_rows, W_text, W_class, scal, scale2d)
    return out[:B, :C]
